# Initial kernel scaffold; baseline (speedup 1.0000x reference)
import jax
import jax.numpy as jnp
from jax.experimental import pallas as pl


def kernel(latents, mus, prototypes):
    raise NotImplementedError("write your pallas kernel here")


# speedup vs baseline: 4.9375x; 4.9375x over previous
import functools

import jax
import jax.numpy as jnp
from jax import lax
from jax.experimental import pallas as pl
from jax.experimental.pallas import tpu as pltpu
from jax.experimental.pallas import tpu_sc as plsc

_K = 1024
_D = 32
_B = 4096
_BETA = 0.25
_EPS = 1e-08
_BT = 512
_NBLK = _B // _BT

_NC = 2
_NS = 16
_NW = _NC * _NS
_BPW = _B // _NW


def _tc_body(l_ref, m_ref, p_ref, idx_ref, loss_ref, pacc_ref, macc_ref):
    i = pl.program_id(0)

    @pl.when(i == 0)
    def _init():
        pacc_ref[...] = jnp.zeros_like(pacc_ref)
        macc_ref[0, 0] = 0.0

    l = l_ref[...]
    m = m_ref[...]
    protos = p_ref[...]

    pn = jnp.sum(protos * protos, axis=1).reshape(1, _K)
    colid = lax.broadcasted_iota(jnp.int32, (_BT, _K), 1)

    ln = jnp.sum(l * l, axis=1, keepdims=True)
    mm1 = lax.dot_general(l, protos, (((1,), (1,)), ((), ())),
                          preferred_element_type=jnp.float32)
    d1 = (ln + pn) - 2.0 * mm1
    rowmin = jnp.min(d1, axis=1, keepdims=True)
    idx = jnp.min(jnp.where(d1 == rowmin, colid, _K), axis=1)
    idx_ref[...] = idx.reshape(1, 1, _BT)

    mn = jnp.sum(m * m, axis=1, keepdims=True)
    mm2 = lax.dot_general(m, protos, (((1,), (1,)), ((), ())),
                          preferred_element_type=jnp.float32)
    d2 = (mn + pn) - 2.0 * mm2

    mse_part = jnp.sum(jnp.where(colid == idx[:, None], d2, 0.0))
    macc_ref[0, 0] += mse_part

    e = jnp.exp(-d2) + _EPS
    rs = jnp.sum(e, axis=1, keepdims=True)
    pacc_ref[...] += jnp.sum(e / rs, axis=0, keepdims=True)

    @pl.when(i == _NBLK - 1)
    def _finish():
        approx = pacc_ref[...] / _B
        ent = -jnp.sum(approx * jnp.log(approx))
        mse_mean = macc_ref[0, 0] / (_B * _D)
        loss = (1.0 + _BETA) * mse_mean + ent
        loss_ref[...] = jnp.full((1, 1), loss, dtype=jnp.float32)


_tc_call = pl.pallas_call(
    _tc_body,
    grid=(_NBLK,),
    in_specs=[
        pl.BlockSpec((_BT, _D), lambda i: (i, 0)),
        pl.BlockSpec((_BT, _D), lambda i: (i, 0)),
        pl.BlockSpec((_K, _D), lambda i: (0, 0)),
    ],
    out_specs=[
        pl.BlockSpec((1, 1, _BT), lambda i: (i, 0, 0)),
        pl.BlockSpec((1, 1), lambda i: (0, 0)),
    ],
    out_shape=[
        jax.ShapeDtypeStruct((_NBLK, 1, _BT), jnp.int32),
        jax.ShapeDtypeStruct((1, 1), jnp.float32),
    ],
    scratch_shapes=[
        pltpu.VMEM((1, _K), jnp.float32),
        pltpu.SMEM((1, 1), jnp.float32),
    ],
    compiler_params=pltpu.CompilerParams(
        dimension_semantics=("arbitrary",),
    ),
)


_DPAD = 128


@functools.cache
def _make_sc_gather():
    @functools.partial(
        pl.kernel,
        out_type=jax.ShapeDtypeStruct((_B, _DPAD), jnp.float32),
        mesh=plsc.VectorSubcoreMesh(core_axis_name="c", subcore_axis_name="s",
                                    num_cores=_NC, num_subcores=_NS),
        scratch_types=[
            pltpu.VMEM((_BPW,), jnp.int32),
            pltpu.VMEM((_BPW, _DPAD), jnp.float32),
            pltpu.SemaphoreType.DMA,
        ],
    )
    def _sc_gather(table_hbm, idx_hbm, out_hbm, idx_v, rows_v, sem):
        wid = lax.axis_index("s") * _NC + lax.axis_index("c")
        base = wid * _BPW
        pltpu.sync_copy(idx_hbm.at[pl.ds(base, _BPW)], idx_v)
        pltpu.async_copy(table_hbm.at[idx_v], rows_v, sem).wait()
        pltpu.sync_copy(rows_v, out_hbm.at[pl.ds(base, _BPW)])

    return _sc_gather


def kernel(latents, mus, prototypes):
    idx3, loss = _tc_call(latents, mus, prototypes)
    idx = idx3.reshape(_B)
    table = jnp.pad(prototypes, ((0, 0), (0, _DPAD - _D)))
    quantized = _make_sc_gather()(table, idx)[:, :_D]
    return quantized, loss.reshape(())


# speedup vs baseline: 5.0537x; 1.0235x over previous
import functools

import jax
import jax.numpy as jnp
from jax import lax
from jax.experimental import pallas as pl
from jax.experimental.pallas import tpu as pltpu
from jax.experimental.pallas import tpu_sc as plsc

_K = 1024
_D = 32
_B = 4096
_BETA = 0.25
_EPS = 1e-08
_BT = 512
_NBLK = _B // _BT

_NC = 2
_NS = 16
_NW = _NC * _NS
_BPW = _B // _NW


def _argmin_body(l_ref, p_ref, idx_ref):
    l = l_ref[...]
    protos = p_ref[...]

    pn = jnp.sum(protos * protos, axis=1).reshape(1, _K)
    ln = jnp.sum(l * l, axis=1, keepdims=True)
    mm1 = lax.dot_general(l, protos, (((1,), (1,)), ((), ())),
                          preferred_element_type=jnp.float32)
    d1 = (ln + pn) - 2.0 * mm1
    rowmin = jnp.min(d1, axis=1, keepdims=True)
    colid_f = lax.broadcasted_iota(jnp.int32, (_BT, _K), 1).astype(jnp.float32)
    idx_f = jnp.min(jnp.where(d1 == rowmin, colid_f, float(_K)), axis=1)
    idx_ref[...] = idx_f.astype(jnp.int32).reshape(1, 1, _BT)


_argmin_call = pl.pallas_call(
    _argmin_body,
    grid=(_NBLK,),
    in_specs=[
        pl.BlockSpec((_BT, _D), lambda i: (i, 0)),
        pl.BlockSpec((_K, _D), lambda i: (0, 0)),
    ],
    out_specs=pl.BlockSpec((1, 1, _BT), lambda i: (i, 0, 0)),
    out_shape=jax.ShapeDtypeStruct((_NBLK, 1, _BT), jnp.int32),
    compiler_params=pltpu.CompilerParams(
        dimension_semantics=("arbitrary",),
    ),
)


def _loss_body(m_ref, p_ref, idx_ref, loss_ref, pacc_ref, macc_ref):
    i = pl.program_id(0)

    @pl.when(i == 0)
    def _init():
        pacc_ref[...] = jnp.zeros_like(pacc_ref)
        macc_ref[0, 0] = 0.0

    m = m_ref[...]
    protos = p_ref[...]
    idx = idx_ref[...].reshape(_BT)

    pn = jnp.sum(protos * protos, axis=1).reshape(1, _K)
    mn = jnp.sum(m * m, axis=1, keepdims=True)
    mm2 = lax.dot_general(m, protos, (((1,), (1,)), ((), ())),
                          preferred_element_type=jnp.float32)
    d2 = (mn + pn) - 2.0 * mm2

    colid = lax.broadcasted_iota(jnp.int32, (_BT, _K), 1)
    mse_part = jnp.sum(jnp.where(colid == idx[:, None], d2, 0.0))
    macc_ref[0, 0] += mse_part

    e = jnp.exp(-d2) + _EPS
    inv_rs = 1.0 / jnp.sum(e, axis=1, keepdims=True)
    pacc_ref[...] += jnp.sum(e * inv_rs, axis=0, keepdims=True)

    @pl.when(i == _NBLK - 1)
    def _finish():
        approx = pacc_ref[...] / _B
        ent = -jnp.sum(approx * jnp.log(approx))
        mse_mean = macc_ref[0, 0] / (_B * _D)
        loss = (1.0 + _BETA) * mse_mean + ent
        loss_ref[...] = jnp.full((1, 1), loss, dtype=jnp.float32)


_loss_call = pl.pallas_call(
    _loss_body,
    grid=(_NBLK,),
    in_specs=[
        pl.BlockSpec((_BT, _D), lambda i: (i, 0)),
        pl.BlockSpec((_K, _D), lambda i: (0, 0)),
        pl.BlockSpec((1, 1, _BT), lambda i: (i, 0, 0)),
    ],
    out_specs=pl.BlockSpec((1, 1), lambda i: (0, 0)),
    out_shape=jax.ShapeDtypeStruct((1, 1), jnp.float32),
    scratch_shapes=[
        pltpu.VMEM((1, _K), jnp.float32),
        pltpu.SMEM((1, 1), jnp.float32),
    ],
    compiler_params=pltpu.CompilerParams(
        dimension_semantics=("arbitrary",),
    ),
)


@functools.cache
def _make_sc_gather():
    @functools.partial(
        pl.kernel,
        out_type=jax.ShapeDtypeStruct((_B, _D), jnp.float32),
        mesh=plsc.VectorSubcoreMesh(core_axis_name="c", subcore_axis_name="s",
                                    num_cores=_NC, num_subcores=_NS),
        scratch_types=[
            pltpu.VMEM((_BPW,), jnp.int32),
            pltpu.VMEM((_BPW, _D), jnp.float32),
            pltpu.SemaphoreType.DMA,
        ],
        compiler_params=pltpu.CompilerParams(use_tc_tiling_on_sc=False),
    )
    def _sc_gather(table_hbm, idx_hbm, out_hbm, idx_v, rows_v, sem):
        wid = lax.axis_index("s") * _NC + lax.axis_index("c")
        base = wid * _BPW
        pltpu.sync_copy(idx_hbm.at[pl.ds(base, _BPW)], idx_v)
        pltpu.async_copy(table_hbm.at[idx_v], rows_v, sem).wait()
        pltpu.sync_copy(rows_v, out_hbm.at[pl.ds(base, _BPW)])

    return _sc_gather


def kernel(latents, mus, prototypes):
    idx3 = _argmin_call(latents, prototypes)
    idx = idx3.reshape(_B)
    quantized = _make_sc_gather()(prototypes, idx)
    loss = _loss_call(mus, prototypes, idx3)
    return quantized, loss.reshape(())


# speedup vs baseline: 6.3081x; 1.2482x over previous
import functools

import jax
import jax.numpy as jnp
from jax import lax
from jax.experimental import pallas as pl
from jax.experimental.pallas import tpu as pltpu
from jax.experimental.pallas import tpu_sc as plsc

_K = 1024
_D = 32
_B = 4096
_BETA = 0.25
_EPS = 1e-08
_BT = 1024
_NBLK = _B // _BT

_NC = 2
_NS = 16
_NW = _NC * _NS
_BPW = _B // _NW


def _argmin_body(lt_ref, pt_ref, idx_ref):
    lt = lt_ref[...]
    pt = pt_ref[...]

    pn = jnp.sum(pt * pt, axis=0).reshape(_K, 1)
    ln = jnp.sum(lt * lt, axis=0, keepdims=True)
    mm1 = lax.dot_general(pt, lt, (((0,), (0,)), ((), ())),
                          preferred_element_type=jnp.float32)
    d1 = (ln + pn) - 2.0 * mm1
    colmin = jnp.min(d1, axis=0, keepdims=True)
    rowid_f = lax.broadcasted_iota(jnp.int32, (_K, _BT), 0).astype(jnp.float32)
    idx_f = jnp.min(jnp.where(d1 == colmin, rowid_f, float(_K)), axis=0)
    idx_ref[...] = idx_f.astype(jnp.int32)


_argmin_call = pl.pallas_call(
    _argmin_body,
    grid=(_NBLK,),
    in_specs=[
        pl.BlockSpec((_D, _BT), lambda i: (0, i)),
        pl.BlockSpec((_D, _K), lambda i: (0, 0)),
    ],
    out_specs=pl.BlockSpec((_BT,), lambda i: (i,)),
    out_shape=jax.ShapeDtypeStruct((_B,), jnp.int32),
    compiler_params=pltpu.CompilerParams(
        dimension_semantics=("arbitrary",),
    ),
)


def _loss_body(mt_ref, pt_ref, idx_ref, loss_ref, pacc_ref, macc_ref):
    i = pl.program_id(0)

    @pl.when(i == 0)
    def _init():
        pacc_ref[...] = jnp.zeros_like(pacc_ref)
        macc_ref[0, 0] = 0.0

    mt = mt_ref[...]
    pt = pt_ref[...]
    idx = idx_ref[...].reshape(1, _BT)

    pn = jnp.sum(pt * pt, axis=0).reshape(_K, 1)
    mn = jnp.sum(mt * mt, axis=0, keepdims=True)
    mm2 = lax.dot_general(pt, mt, (((0,), (0,)), ((), ())),
                          preferred_element_type=jnp.float32)
    d2 = (mn + pn) - 2.0 * mm2

    rowid = lax.broadcasted_iota(jnp.int32, (_K, _BT), 0)
    mse_part = jnp.sum(jnp.where(rowid == idx, d2, 0.0))
    macc_ref[0, 0] += mse_part

    e = jnp.exp(-d2) + _EPS
    inv_rs = 1.0 / jnp.sum(e, axis=0, keepdims=True)
    probs = e * inv_rs
    ones = jnp.ones((_BT, 1), dtype=jnp.float32)
    pacc_ref[...] += lax.dot_general(probs, ones, (((1,), (0,)), ((), ())),
                                     preferred_element_type=jnp.float32)

    @pl.when(i == _NBLK - 1)
    def _finish():
        approx = pacc_ref[...] / _B
        ent = -jnp.sum(approx * jnp.log(approx))
        mse_mean = macc_ref[0, 0] / (_B * _D)
        loss = (1.0 + _BETA) * mse_mean + ent
        loss_ref[...] = jnp.full((1, 1), loss, dtype=jnp.float32)


_loss_call = pl.pallas_call(
    _loss_body,
    grid=(_NBLK,),
    in_specs=[
        pl.BlockSpec((_D, _BT), lambda i: (0, i)),
        pl.BlockSpec((_D, _K), lambda i: (0, 0)),
        pl.BlockSpec((_BT,), lambda i: (i,)),
    ],
    out_specs=pl.BlockSpec((1, 1), lambda i: (0, 0)),
    out_shape=jax.ShapeDtypeStruct((1, 1), jnp.float32),
    scratch_shapes=[
        pltpu.VMEM((_K, 1), jnp.float32),
        pltpu.SMEM((1, 1), jnp.float32),
    ],
    compiler_params=pltpu.CompilerParams(
        dimension_semantics=("arbitrary",),
    ),
)


@functools.cache
def _make_sc_gather():
    @functools.partial(
        pl.kernel,
        out_type=jax.ShapeDtypeStruct((_B, _D), jnp.float32),
        mesh=plsc.VectorSubcoreMesh(core_axis_name="c", subcore_axis_name="s",
                                    num_cores=_NC, num_subcores=_NS),
        scratch_types=[
            pltpu.VMEM((_BPW,), jnp.int32),
            pltpu.VMEM((_BPW, _D), jnp.float32),
            pltpu.SemaphoreType.DMA,
        ],
        compiler_params=pltpu.CompilerParams(use_tc_tiling_on_sc=False),
    )
    def _sc_gather(table_hbm, idx_hbm, out_hbm, idx_v, rows_v, sem):
        wid = lax.axis_index("s") * _NC + lax.axis_index("c")
        base = wid * _BPW
        pltpu.sync_copy(idx_hbm.at[pl.ds(base, _BPW)], idx_v)
        pltpu.async_copy(table_hbm.at[idx_v], rows_v, sem).wait()
        pltpu.sync_copy(rows_v, out_hbm.at[pl.ds(base, _BPW)])

    return _sc_gather


def kernel(latents, mus, prototypes):
    lt = latents.T
    mt = mus.T
    pt = prototypes.T
    idx = _argmin_call(lt, pt)
    quantized = _make_sc_gather()(prototypes, idx)
    loss = _loss_call(mt, pt, idx)
    return quantized, loss.reshape(())


# speedup vs baseline: 6.6697x; 1.0573x over previous
import functools

import jax
import jax.numpy as jnp
from jax import lax
from jax.experimental import pallas as pl
from jax.experimental.pallas import tpu as pltpu
from jax.experimental.pallas import tpu_sc as plsc

_K = 1024
_D = 32
_B = 4096
_BETA = 0.25
_EPS = 1e-08
_BT = 2048
_NBLK = _B // _BT

_NC = 2
_NS = 16
_NW = _NC * _NS
_BPW = _B // _NW


def _argmin_body(lt_ref, pt_ref, idx_ref):
    lt = lt_ref[...]
    pt = pt_ref[...]

    pn = jnp.sum(pt * pt, axis=0).reshape(_K, 1)
    ln = jnp.sum(lt * lt, axis=0, keepdims=True)
    mm1 = lax.dot_general(pt, lt, (((0,), (0,)), ((), ())),
                          preferred_element_type=jnp.float32)
    d1 = (ln + pn) - 2.0 * mm1
    colmin = jnp.min(d1, axis=0, keepdims=True)
    rowid_f = lax.broadcasted_iota(jnp.int32, (_K, _BT), 0).astype(jnp.float32)
    idx_f = jnp.min(jnp.where(d1 == colmin, rowid_f, float(_K)), axis=0)
    idx_ref[...] = idx_f.astype(jnp.int32)


_argmin_call = pl.pallas_call(
    _argmin_body,
    grid=(_NBLK,),
    in_specs=[
        pl.BlockSpec((_D, _BT), lambda i: (0, i)),
        pl.BlockSpec((_D, _K), lambda i: (0, 0)),
    ],
    out_specs=pl.BlockSpec((_BT,), lambda i: (i,)),
    out_shape=jax.ShapeDtypeStruct((_B,), jnp.int32),
    compiler_params=pltpu.CompilerParams(
        dimension_semantics=("arbitrary",),
    ),
)


def _loss_body(mt_ref, pt_ref, idx_ref, loss_ref, pacc_ref, macc_ref):
    i = pl.program_id(0)

    @pl.when(i == 0)
    def _init():
        pacc_ref[...] = jnp.zeros_like(pacc_ref)
        macc_ref[0, 0] = 0.0

    mt = mt_ref[...]
    pt = pt_ref[...]
    idx = idx_ref[...].reshape(1, _BT)

    pn = jnp.sum(pt * pt, axis=0, keepdims=True)
    mn = jnp.sum(mt * mt, axis=0, keepdims=True)
    onesk = jnp.ones((1, _K), dtype=jnp.float32)
    onesb = jnp.ones((1, _BT), dtype=jnp.float32)
    lhs = jnp.concatenate([pt * -2.0, pn, onesk], axis=0)
    rhs = jnp.concatenate([mt, onesb, mn], axis=0)
    d2 = lax.dot_general(lhs, rhs, (((0,), (0,)), ((), ())),
                         preferred_element_type=jnp.float32)

    rowid = lax.broadcasted_iota(jnp.int32, (_K, _BT), 0)
    mse_part = jnp.sum(jnp.where(rowid == idx, d2, 0.0))
    macc_ref[0, 0] += mse_part

    e = jnp.exp(-d2) + _EPS
    inv_rs = 1.0 / jnp.sum(e, axis=0, keepdims=True)
    probs = e * inv_rs
    ones = jnp.ones((_BT, 1), dtype=jnp.float32)
    pacc_ref[...] += lax.dot_general(probs, ones, (((1,), (0,)), ((), ())),
                                     preferred_element_type=jnp.float32)

    @pl.when(i == _NBLK - 1)
    def _finish():
        approx = pacc_ref[...] / _B
        ent = -jnp.sum(approx * jnp.log(approx))
        mse_mean = macc_ref[0, 0] / (_B * _D)
        loss = (1.0 + _BETA) * mse_mean + ent
        loss_ref[...] = jnp.full((1, 1), loss, dtype=jnp.float32)


_loss_call = pl.pallas_call(
    _loss_body,
    grid=(_NBLK,),
    in_specs=[
        pl.BlockSpec((_D, _BT), lambda i: (0, i)),
        pl.BlockSpec((_D, _K), lambda i: (0, 0)),
        pl.BlockSpec((_BT,), lambda i: (i,)),
    ],
    out_specs=pl.BlockSpec((1, 1), lambda i: (0, 0)),
    out_shape=jax.ShapeDtypeStruct((1, 1), jnp.float32),
    scratch_shapes=[
        pltpu.VMEM((_K, 1), jnp.float32),
        pltpu.SMEM((1, 1), jnp.float32),
    ],
    compiler_params=pltpu.CompilerParams(
        dimension_semantics=("arbitrary",),
    ),
)


@functools.cache
def _make_sc_gather():
    @functools.partial(
        pl.kernel,
        out_type=jax.ShapeDtypeStruct((_B, _D), jnp.float32),
        mesh=plsc.VectorSubcoreMesh(core_axis_name="c", subcore_axis_name="s",
                                    num_cores=_NC, num_subcores=_NS),
        scratch_types=[
            pltpu.VMEM((_BPW,), jnp.int32),
            pltpu.VMEM((_BPW, _D), jnp.float32),
            pltpu.SemaphoreType.DMA,
        ],
        compiler_params=pltpu.CompilerParams(use_tc_tiling_on_sc=False),
    )
    def _sc_gather(table_hbm, idx_hbm, out_hbm, idx_v, rows_v, sem):
        wid = lax.axis_index("s") * _NC + lax.axis_index("c")
        base = wid * _BPW
        pltpu.sync_copy(idx_hbm.at[pl.ds(base, _BPW)], idx_v)
        pltpu.async_copy(table_hbm.at[idx_v], rows_v, sem).wait()
        pltpu.sync_copy(rows_v, out_hbm.at[pl.ds(base, _BPW)])

    return _sc_gather


def kernel(latents, mus, prototypes):
    lt = latents.T
    mt = mus.T
    pt = prototypes.T
    idx = _argmin_call(lt, pt)
    quantized = _make_sc_gather()(prototypes, idx)
    loss = _loss_call(mt, pt, idx)
    return quantized, loss.reshape(())


# speedup vs baseline: 7.2538x; 1.0876x over previous
import functools

import jax
import jax.numpy as jnp
from jax import lax
from jax.experimental import pallas as pl
from jax.experimental.pallas import tpu as pltpu
from jax.experimental.pallas import tpu_sc as plsc

_K = 1024
_D = 32
_B = 4096
_BETA = 0.25
_EPS = 1e-08
_BT = 2048
_NBLK = _B // _BT

_NC = 2
_NS = 16
_NW = _NC * _NS
_BPW = _B // _NW


def _argmin_body(lt_ref, pt_ref, idx_ref):
    lt = lt_ref[...]
    pt = pt_ref[...]

    pn = jnp.sum(pt * pt, axis=0).reshape(_K, 1)
    ln = jnp.sum(lt * lt, axis=0, keepdims=True)
    mm1 = lax.dot_general(pt, lt, (((0,), (0,)), ((), ())),
                          preferred_element_type=jnp.float32)
    d1 = (ln + pn) - 2.0 * mm1
    colmin = jnp.min(d1, axis=0, keepdims=True)
    rowid_f = lax.broadcasted_iota(jnp.int32, (_K, _BT), 0).astype(jnp.float32)
    idx_f = jnp.min(jnp.where(d1 == colmin, rowid_f, float(_K)), axis=0)
    idx_ref[...] = idx_f.astype(jnp.int32)


_argmin_call = pl.pallas_call(
    _argmin_body,
    grid=(_NBLK,),
    in_specs=[
        pl.BlockSpec((_D, _BT), lambda i: (0, i)),
        pl.BlockSpec((_D, _K), lambda i: (0, 0)),
    ],
    out_specs=pl.BlockSpec((_BT,), lambda i: (i,)),
    out_shape=jax.ShapeDtypeStruct((_B,), jnp.int32),
    compiler_params=pltpu.CompilerParams(
        dimension_semantics=("arbitrary",),
    ),
)


def _loss_body(mt_ref, pt_ref, idx_ref, loss_ref, pacc_ref, macc_ref):
    i = pl.program_id(0)

    @pl.when(i == 0)
    def _init():
        pacc_ref[...] = jnp.zeros_like(pacc_ref)
        macc_ref[0, 0] = 0.0

    mt = mt_ref[...]
    pt = pt_ref[...]
    idx = idx_ref[...].reshape(1, _BT)

    pn = jnp.sum(pt * pt, axis=0, keepdims=True)
    mn = jnp.sum(mt * mt, axis=0, keepdims=True)
    onesk = jnp.ones((1, _K), dtype=jnp.float32)
    onesb = jnp.ones((1, _BT), dtype=jnp.float32)
    lhs = jnp.concatenate([pt * -2.0, pn, onesk], axis=0)
    rhs = jnp.concatenate([mt, onesb, mn], axis=0)
    d2 = lax.dot_general(lhs, rhs, (((0,), (0,)), ((), ())),
                         preferred_element_type=jnp.float32)

    rowid = lax.broadcasted_iota(jnp.int32, (_K, _BT), 0)
    mse_part = jnp.sum(jnp.where(rowid == idx, d2, 0.0))
    macc_ref[0, 0] += mse_part

    e = jnp.exp(-d2) + _EPS
    inv_rs = 1.0 / jnp.sum(e, axis=0, keepdims=True)
    probs = e * inv_rs
    ones = jnp.ones((_BT, 1), dtype=jnp.float32)
    pacc_ref[...] += lax.dot_general(probs, ones, (((1,), (0,)), ((), ())),
                                     preferred_element_type=jnp.float32)

    @pl.when(i == _NBLK - 1)
    def _finish():
        approx = pacc_ref[...] / _B
        ent = -jnp.sum(approx * jnp.log(approx))
        mse_mean = macc_ref[0, 0] / (_B * _D)
        loss = (1.0 + _BETA) * mse_mean + ent
        loss_ref[...] = jnp.full((1, 1), loss, dtype=jnp.float32)


_loss_call = pl.pallas_call(
    _loss_body,
    grid=(_NBLK,),
    in_specs=[
        pl.BlockSpec((_D, _BT), lambda i: (0, i)),
        pl.BlockSpec((_D, _K), lambda i: (0, 0)),
        pl.BlockSpec((_BT,), lambda i: (i,)),
    ],
    out_specs=pl.BlockSpec((1, 1), lambda i: (0, 0)),
    out_shape=jax.ShapeDtypeStruct((1, 1), jnp.float32),
    scratch_shapes=[
        pltpu.VMEM((_K, 1), jnp.float32),
        pltpu.SMEM((1, 1), jnp.float32),
    ],
    compiler_params=pltpu.CompilerParams(
        dimension_semantics=("arbitrary",),
    ),
)


_DPAD = 128


@functools.cache
def _make_sc_gather():
    @functools.partial(
        pl.kernel,
        out_type=jax.ShapeDtypeStruct((_D, _B), jnp.float32),
        mesh=plsc.VectorSubcoreMesh(core_axis_name="c", subcore_axis_name="s",
                                    num_cores=_NC, num_subcores=_NS),
        scratch_types=[
            pltpu.VMEM((_BPW,), jnp.int32),
            pltpu.VMEM((_BPW, _DPAD), jnp.float32),
            pltpu.VMEM((_D, _BPW), jnp.float32),
            pltpu.SemaphoreType.DMA,
        ],
        compiler_params=pltpu.CompilerParams(needs_layout_passes=False),
    )
    def _sc_gather(table_hbm, idx_hbm, out_hbm, idx_v, rows_v, outt_v, sem):
        wid = lax.axis_index("s") * _NC + lax.axis_index("c")
        base = wid * _BPW
        pltpu.sync_copy(idx_hbm.at[pl.ds(base, _BPW)], idx_v)
        pltpu.async_copy(table_hbm.at[idx_v], rows_v, sem).wait()
        lanes = lax.iota(jnp.int32, 16)
        for c in range(_BPW // 16):
            rows16 = lanes + (c * 16)
            for d in range(_D):
                vals = plsc.load_gather(
                    rows_v, [rows16, jnp.full((16,), d, jnp.int32)])
                outt_v[d, pl.ds(c * 16, 16)] = vals
        pltpu.sync_copy(outt_v, out_hbm.at[:, pl.ds(base, _BPW)])

    return _sc_gather


def kernel(latents, mus, prototypes):
    lt = latents.T
    mt = mus.T
    pt = prototypes.T
    idx = _argmin_call(lt, pt)
    table = jnp.pad(prototypes, ((0, 0), (0, _DPAD - _D)))
    quantized_t = _make_sc_gather()(table, idx)
    loss = _loss_call(mt, pt, idx)
    return quantized_t.T, loss.reshape(())
